# baseline (speedup 1.0000x reference)
import jax
import jax.numpy as jnp
from jax import lax
from jax.experimental import pallas as pl
from jax.experimental.pallas import tpu as pltpu
from jax.experimental.pallas import tpu_sc as plsc

BATCH = 16384
N_FIELDS = 26
VOCAB = 1_000_000
NUM_IDS = BATCH * N_FIELDS
NUM_WORKERS = 32
ROWS_PER_W = BATCH // NUM_WORKERS
IDS_PER_W = ROWS_PER_W * N_FIELDS
LANES = 16
CHUNKS = ROWS_PER_W // LANES


def _flat_body(w_ref, ow_ref):
    ow_ref[...] = w_ref[0, :]


def _flatten_w(wt):
    return pl.pallas_call(
        _flat_body,
        in_specs=[pl.BlockSpec((1, VOCAB), lambda: (0, 0))],
        out_specs=pl.BlockSpec((VOCAB,), lambda: (0,)),
        out_shape=jax.ShapeDtypeStruct((VOCAB,), jnp.float32),
    )(wt)


_FSPLIT = 13
_NA = _FSPLIT * ROWS_PER_W
_NB = IDS_PER_W - _NA


def _sc_body(
    idx_hbm, w_hbm, bias_hbm, out_hbm, idx_a, idx_b, g_a, g_b, bias_v, acc_v,
    sem, sem_a, sem_b
):
    wid = lax.axis_index("s") * 2 + lax.axis_index("c")
    row0 = wid * ROWS_PER_W

    def idx_copy(f):
        dst = idx_a if f < _FSPLIT else idx_b
        fo = f if f < _FSPLIT else f - _FSPLIT
        return pltpu.async_copy(
            idx_hbm.at[pl.ds(f * BATCH + row0, ROWS_PER_W)],
            dst.at[pl.ds(fo * ROWS_PER_W, ROWS_PER_W)],
            sem,
        )

    for f in range(N_FIELDS):
        idx_copy(f).start()
    pltpu.sync_copy(bias_hbm, bias_v)
    for f in range(N_FIELDS):
        idx_copy(f).wait()

    ga = pltpu.async_copy(w_hbm.at[idx_a], g_a, sem_a)
    gb = pltpu.async_copy(w_hbm.at[idx_b], g_b, sem_b)
    ga.start()
    gb.start()

    zeros = jnp.zeros((LANES,), jnp.int32)
    bvec = plsc.load_gather(bias_v, [zeros])
    ga.wait()
    gb.wait()

    def chunk_body(c, _):
        b0 = c * LANES
        acc = bvec
        for f in range(_FSPLIT):
            acc = acc + g_a[pl.ds(f * ROWS_PER_W + b0, LANES)]
        for f in range(N_FIELDS - _FSPLIT):
            acc = acc + g_b[pl.ds(f * ROWS_PER_W + b0, LANES)]
        acc_v[pl.ds(b0, LANES)] = acc
        return 0

    lax.fori_loop(0, CHUNKS, chunk_body, 0)

    pltpu.sync_copy(acc_v, out_hbm.at[pl.ds(row0, ROWS_PER_W)])


@jax.jit
def _sc_call(idx_flat, w_flat, bias):
    mesh = plsc.VectorSubcoreMesh(core_axis_name="c", subcore_axis_name="s")
    fn = pl.kernel(
        _sc_body,
        out_type=jax.ShapeDtypeStruct((BATCH,), jnp.float32),
        mesh=mesh,
        compiler_params=pltpu.CompilerParams(needs_layout_passes=False),
        scratch_types=[
            pltpu.VMEM((_NA,), jnp.int32),
            pltpu.VMEM((_NB,), jnp.int32),
            pltpu.VMEM((_NA,), jnp.float32),
            pltpu.VMEM((_NB,), jnp.float32),
            pltpu.VMEM((1,), jnp.float32),
            pltpu.VMEM((ROWS_PER_W,), jnp.float32),
            pltpu.SemaphoreType.DMA,
            pltpu.SemaphoreType.DMA,
            pltpu.SemaphoreType.DMA,
        ],
    )
    return fn(idx_flat, w_flat, bias)


def kernel(indices, weights, bias):
    w_flat = _flatten_w(lax.transpose(weights, (1, 0)))
    out = _sc_call(indices.T.reshape(-1), w_flat, bias)
    return out.reshape(BATCH, 1)


# speedup vs baseline: 1.2335x; 1.2335x over previous
import jax
import jax.numpy as jnp
from jax import lax
from jax.experimental import pallas as pl
from jax.experimental.pallas import tpu as pltpu
from jax.experimental.pallas import tpu_sc as plsc

BATCH = 16384
N_FIELDS = 26
VOCAB = 1_000_000
NUM_IDS = BATCH * N_FIELDS
NUM_WORKERS = 32
ROWS_PER_W = BATCH // NUM_WORKERS
IDS_PER_W = ROWS_PER_W * N_FIELDS
LANES = 16
CHUNKS = ROWS_PER_W // LANES


def _flat_body(w_ref, ow_ref):
    ow_ref[...] = w_ref[0, :]


def _flatten_w(wt):
    return pl.pallas_call(
        _flat_body,
        in_specs=[pl.BlockSpec((1, VOCAB), lambda: (0, 0))],
        out_specs=pl.BlockSpec((VOCAB,), lambda: (0,)),
        out_shape=jax.ShapeDtypeStruct((VOCAB,), jnp.float32),
    )(wt)


def _sc_body(idx_hbm, w_hbm, bias_hbm, out_hbm, idx_v, g_v, bias_v, acc_v, sem, gsem):
    wid = lax.axis_index("s") * 2 + lax.axis_index("c")
    row0 = wid * ROWS_PER_W

    def idx_copy(f):
        return pltpu.async_copy(
            idx_hbm.at[pl.ds(f * BATCH + row0, ROWS_PER_W)],
            idx_v.at[pl.ds(f * ROWS_PER_W, ROWS_PER_W)],
            sem,
        )

    for f in range(N_FIELDS):
        idx_copy(f).start()
    pltpu.sync_copy(bias_hbm, bias_v)
    for f in range(N_FIELDS):
        idx_copy(f).wait()

    pltpu.async_copy(w_hbm.at[idx_v], g_v, gsem).wait()

    zeros = jnp.zeros((LANES,), jnp.int32)
    bvec = plsc.load_gather(bias_v, [zeros])

    def chunk_body(c, _):
        b0 = c * LANES
        acc = bvec
        for f in range(N_FIELDS):
            acc = acc + g_v[pl.ds(f * ROWS_PER_W + b0, LANES)]
        acc_v[pl.ds(b0, LANES)] = acc
        return 0

    lax.fori_loop(0, CHUNKS, chunk_body, 0)

    pltpu.sync_copy(acc_v, out_hbm.at[pl.ds(row0, ROWS_PER_W)])


@jax.jit
def _sc_call(idx_flat, w_flat, bias):
    mesh = plsc.VectorSubcoreMesh(core_axis_name="c", subcore_axis_name="s")
    fn = pl.kernel(
        _sc_body,
        out_type=jax.ShapeDtypeStruct((BATCH,), jnp.float32),
        mesh=mesh,
        compiler_params=pltpu.CompilerParams(needs_layout_passes=False),
        scratch_types=[
            pltpu.VMEM((IDS_PER_W,), jnp.int32),
            pltpu.VMEM((IDS_PER_W,), jnp.float32),
            pltpu.VMEM((1,), jnp.float32),
            pltpu.VMEM((ROWS_PER_W,), jnp.float32),
            pltpu.SemaphoreType.DMA,
            pltpu.SemaphoreType.DMA,
        ],
    )
    return fn(idx_flat, w_flat, bias)


def kernel(indices, weights, bias):
    w_flat = _flatten_w(lax.transpose(weights, (1, 0)))
    out = _sc_call(indices.T.reshape(-1), w_flat, bias)
    return out.reshape(BATCH, 1)


# speedup vs baseline: 1.2662x; 1.0266x over previous
import jax
import jax.numpy as jnp
from jax import lax
from jax.experimental import pallas as pl
from jax.experimental.pallas import tpu as pltpu
from jax.experimental.pallas import tpu_sc as plsc

BATCH = 16384
N_FIELDS = 26
VOCAB = 1_000_000
NUM_IDS = BATCH * N_FIELDS
NUM_WORKERS = 32
ROWS_PER_W = BATCH // NUM_WORKERS
IDS_PER_W = ROWS_PER_W * N_FIELDS
LANES = 16
CHUNKS = ROWS_PER_W // LANES


def _flat_body(w_ref, ow_ref):
    ow_ref[...] = w_ref[0, :]


def _flatten_w(wt):
    return pl.pallas_call(
        _flat_body,
        in_specs=[pl.BlockSpec((1, VOCAB), lambda: (0, 0))],
        out_specs=pl.BlockSpec((VOCAB,), lambda: (0,)),
        out_shape=jax.ShapeDtypeStruct((VOCAB,), jnp.float32),
    )(wt)


def _sc_body(idx_hbm, w_hbm, bias_hbm, out_hbm, idx_v, g_v, bias_v, acc_v, sem):
    wid = lax.axis_index("s") * 2 + lax.axis_index("c")
    row0 = wid * ROWS_PER_W

    def idx_copy(f):
        return pltpu.async_copy(
            idx_hbm.at[pl.ds(f * BATCH + row0, ROWS_PER_W)],
            idx_v.at[pl.ds(f * ROWS_PER_W, ROWS_PER_W)],
            sem,
        )

    for f in range(N_FIELDS):
        idx_copy(f).start()
    pltpu.sync_copy(bias_hbm, bias_v)
    for f in range(N_FIELDS):
        idx_copy(f).wait()

    pltpu.async_copy(w_hbm.at[idx_v], g_v, sem).wait()

    zeros = jnp.zeros((LANES,), jnp.int32)
    bvec = plsc.load_gather(bias_v, [zeros])

    def chunk_body(c, _):
        b0 = c * LANES
        acc = bvec
        for f in range(N_FIELDS):
            acc = acc + g_v[pl.ds(f * ROWS_PER_W + b0, LANES)]
        acc_v[pl.ds(b0, LANES)] = acc
        return 0

    lax.fori_loop(0, CHUNKS, chunk_body, 0)

    pltpu.sync_copy(acc_v, out_hbm.at[pl.ds(row0, ROWS_PER_W)])


@jax.jit
def _sc_call(idx_flat, w_flat, bias):
    mesh = plsc.VectorSubcoreMesh(core_axis_name="c", subcore_axis_name="s")
    fn = pl.kernel(
        _sc_body,
        out_type=jax.ShapeDtypeStruct((BATCH,), jnp.float32),
        mesh=mesh,
        compiler_params=pltpu.CompilerParams(needs_layout_passes=False),
        scratch_types=[
            pltpu.VMEM((IDS_PER_W,), jnp.int32),
            pltpu.VMEM((IDS_PER_W,), jnp.float32),
            pltpu.VMEM((1,), jnp.float32),
            pltpu.VMEM((ROWS_PER_W,), jnp.float32),
            pltpu.SemaphoreType.DMA,
        ],
    )
    return fn(idx_flat, w_flat, bias)


def kernel(indices, weights, bias):
    w_flat = _flatten_w(lax.transpose(weights, (1, 0)))
    out = _sc_call(indices.T.reshape(-1), w_flat, bias)
    return out.reshape(BATCH, 1)
